# baseline (speedup 1.0000x reference)
import functools
import itertools
import math

import jax
import jax.numpy as jnp
from jax import lax
from jax.experimental import pallas as pl
from jax.experimental.pallas import tpu as pltpu
from jax.experimental.pallas import tpu_sc as plsc

N = 65536
B = 1024
NP = N + B
NT = NP // B
CMID = 32
EPS = 1e-5

_OFFS = list(itertools.product([-1, 0, 1], repeat=3))


def _sel(pred):
    return [i for i, o in enumerate(_OFFS) if pred(o)]


_K133 = _sel(lambda o: o[0] == 0)
_K313 = _sel(lambda o: o[1] == 0)
_K311 = _sel(lambda o: o[1] == 0 and o[2] == 0)
_K131 = _sel(lambda o: o[0] == 0 and o[2] == 0)
_K113 = _sel(lambda o: o[0] == 0 and o[1] == 0)
_K333 = list(range(27))

F32 = jnp.float32
BF16 = jnp.bfloat16


def _lrelu(t):
    return jnp.maximum(t, 0.01 * t)


def _flat_w(w_list):
    k = len(w_list)
    kg = -(-k // 4)
    pads = [jnp.zeros_like(w_list[0])] * (4 * kg - k)
    return jnp.concatenate(list(w_list) + pads, axis=1)


def _tc1(x, Wf):
    kg = Wf.shape[1] // 128

    def body(x_ref, w_ref, o_ref):
        i = pl.program_id(0)

        @pl.when(i < NT - 1)
        def _():
            y = jnp.dot(x_ref[...], w_ref[...], preferred_element_type=F32)
            for p in range(kg):
                o_ref[p] = y[:, 128 * p:128 * (p + 1)].astype(BF16)

        @pl.when(i == NT - 1)
        def _():
            o_ref[...] = jnp.zeros_like(o_ref)

    return pl.pallas_call(
        body,
        grid=(NT,),
        in_specs=[
            pl.BlockSpec((B, 16), lambda i: (jnp.minimum(i, NT - 2), 0)),
            pl.BlockSpec(Wf.shape, lambda i: (0, 0)),
        ],
        out_specs=pl.BlockSpec((kg, B, 128), lambda i: (0, i, 0)),
        out_shape=jax.ShapeDtypeStruct((kg, NP, 128), BF16),
    )(x, Wf)


def _tc2(Aa, Wf12, Wf3, P):
    kg0 = Wf12.shape[1] // 128
    kg = kg0 + Wf3.shape[1] // 128

    def body(a_ref, w12_ref, w3_ref, p_ref, o_ref):
        i = pl.program_id(0)

        @pl.when(i < NT - 1)
        def _():
            u0 = _lrelu(a_ref[:, 0:CMID].astype(F32)) * p_ref[0] + p_ref[1]
            u1 = _lrelu(a_ref[:, CMID:2 * CMID].astype(F32)) * p_ref[2] + p_ref[3]
            y0 = jnp.dot(u0, w12_ref[...], preferred_element_type=F32)
            y1 = jnp.dot(u1, w3_ref[...], preferred_element_type=F32)
            for p in range(kg0):
                o_ref[p] = y0[:, 128 * p:128 * (p + 1)].astype(BF16)
            for p in range(kg - kg0):
                o_ref[kg0 + p] = y1[:, 128 * p:128 * (p + 1)].astype(BF16)

        @pl.when(i == NT - 1)
        def _():
            o_ref[...] = jnp.zeros_like(o_ref)

    return pl.pallas_call(
        body,
        grid=(NT,),
        in_specs=[
            pl.BlockSpec((B, 128), lambda i: (i, 0)),
            pl.BlockSpec(Wf12.shape, lambda i: (0, 0)),
            pl.BlockSpec(Wf3.shape, lambda i: (0, 0)),
            pl.BlockSpec((4, 1, CMID), lambda i: (0, 0, 0)),
        ],
        out_specs=pl.BlockSpec((kg, B, 128), lambda i: (0, i, 0)),
        out_shape=jax.ShapeDtypeStruct((kg, NP, 128), BF16),
    )(Aa, Wf12, Wf3, P)


def _tc3(Ab, Wfr, P):
    kg = Wfr.shape[1] // 128

    def body(a_ref, w_ref, p_ref, ra_ref, o_ref):
        i = pl.program_id(0)

        @pl.when(i < NT - 1)
        def _():
            rA = ((_lrelu(a_ref[:, 0:CMID].astype(F32)) * p_ref[0] + p_ref[1])
                  + (_lrelu(a_ref[:, CMID:2 * CMID].astype(F32)) * p_ref[2]
                     + p_ref[3]))
            ra_ref[...] = rA
            y = jnp.dot(rA, w_ref[...], preferred_element_type=F32)
            for p in range(kg):
                o_ref[p] = y[:, 128 * p:128 * (p + 1)].astype(BF16)

        @pl.when(i == NT - 1)
        def _():
            ra_ref[...] = jnp.zeros_like(ra_ref)
            o_ref[...] = jnp.zeros_like(o_ref)

    return pl.pallas_call(
        body,
        grid=(NT,),
        in_specs=[
            pl.BlockSpec((B, 128), lambda i: (i, 0)),
            pl.BlockSpec(Wfr.shape, lambda i: (0, 0)),
            pl.BlockSpec((4, 1, CMID), lambda i: (0, 0, 0)),
        ],
        out_specs=[
            pl.BlockSpec((B, CMID), lambda i: (i, 0)),
            pl.BlockSpec((kg, B, 128), lambda i: (0, i, 0)),
        ],
        out_shape=[
            jax.ShapeDtypeStruct((NP, CMID), F32),
            jax.ShapeDtypeStruct((kg, NP, 128), BF16),
        ],
    )(Ab, Wfr, P)


def _tc4(Rc, rA, Wfd, P):
    kg = Wfd.shape[1] // 128

    def body(r_ref, ra_ref, w_ref, p_ref, o_ref):
        i = pl.program_id(0)

        @pl.when(i < NT - 1)
        def _():
            s = (jax.nn.sigmoid(r_ref[:, 0:CMID].astype(F32) * p_ref[0] + p_ref[1])
                 + jax.nn.sigmoid(r_ref[:, CMID:2 * CMID].astype(F32) * p_ref[2] + p_ref[3])
                 + jax.nn.sigmoid(r_ref[:, 2 * CMID:3 * CMID].astype(F32) * p_ref[4] + p_ref[5]))
            recon = s * ra_ref[...]
            y = jnp.dot(recon, w_ref[...], preferred_element_type=F32)
            for p in range(kg):
                o_ref[p] = y[:, 128 * p:128 * (p + 1)].astype(BF16)

        @pl.when(i == NT - 1)
        def _():
            o_ref[...] = jnp.zeros_like(o_ref)

    return pl.pallas_call(
        body,
        grid=(NT,),
        in_specs=[
            pl.BlockSpec((B, 128), lambda i: (i, 0)),
            pl.BlockSpec((B, CMID), lambda i: (i, 0)),
            pl.BlockSpec(Wfd.shape, lambda i: (0, 0)),
            pl.BlockSpec((6, 1, CMID), lambda i: (0, 0, 0)),
        ],
        out_specs=pl.BlockSpec((kg, B, 128), lambda i: (0, i, 0)),
        out_shape=jax.ShapeDtypeStruct((kg, NP, 128), BF16),
    )(Rc, rA, Wfd, P)


NWORK = 32
RW = N // NWORK
CH = 128
NCH = RW // CH


@functools.lru_cache(maxsize=None)
def _make_sc_gather(k, G, KG):
    g = k // G
    mesh = plsc.VectorSubcoreMesh(core_axis_name="c", subcore_axis_name="s")

    @functools.partial(
        pl.kernel,
        out_type=jax.ShapeDtypeStruct((NP, 128), BF16),
        mesh=mesh,
        scratch_types=[
            pltpu.VMEM((k, CH), jnp.int32),
            pltpu.VMEM((k, CH), jnp.int32),
            pltpu.VMEM((G, CH, CMID), BF16),
            pltpu.VMEM((G, CH, CMID), BF16),
            pltpu.SemaphoreType.DMA,
            pltpu.SemaphoreType.DMA,
            pltpu.SemaphoreType.DMA,
            pltpu.SemaphoreType.DMA,
            pltpu.SemaphoreType.DMA,
            pltpu.SemaphoreType.DMA,
        ],
        compiler_params=pltpu.CompilerParams(use_tc_tiling_on_sc=False),
    )
    def kfn(tbl, idx4, out, idx_a, idx_b, acc_a, acc_b,
            sem_ia, sem_ib, sem_ga, sem_gb, sem_sa, sem_sb):
        wid = lax.axis_index("s") * 2 + lax.axis_index("c")
        zero32 = jnp.zeros((CMID,), BF16)

        def zero_acc(acc):
            def zbody(r, carry):
                for grp in range(G):
                    acc[grp, r, :] = zero32
                return carry
            lax.fori_loop(0, CH, zbody, 0)

        def drain_store(acc, sem_s):
            for grp in range(G):
                pltpu.make_async_copy(
                    acc.at[grp],
                    out.at[pl.ds(0, CH), pl.ds(CMID * grp, CMID)], sem_s).wait()

        def fire_phase(c, idx_v, acc, sem_i, sem_g, sem_s, first):
            @pl.when(jnp.logical_not(first))
            def _():
                drain_store(acc, sem_s)
            pltpu.make_async_copy(idx4.at[wid, 0], idx_v, sem_i).wait()
            zero_acc(acc)
            for grp in range(G):
                for j in range(g):
                    pltpu.async_copy(
                        tbl.at[idx_v.at[grp * g + j]], acc.at[grp], sem_g,
                        add=True)

        def finish_phase(c, idx_v, acc, sem_i, sem_g, sem_s):
            for grp in range(G):
                for j in range(g):
                    pltpu.make_async_copy(
                        tbl.at[idx_v.at[grp * g + j]], acc.at[grp],
                        sem_g).wait()
            base = wid * RW + c * CH
            for grp in range(G):
                pltpu.async_copy(
                    acc.at[grp],
                    out.at[pl.ds(base, CH), pl.ds(CMID * grp, CMID)], sem_s)
            @pl.when(c + 2 < NCH)
            def _():
                pltpu.async_copy(idx4.at[wid, c + 2], idx_v, sem_i)

        pltpu.async_copy(idx4.at[wid, 0], idx_a, sem_ia)
        pltpu.async_copy(idx4.at[wid, 1], idx_b, sem_ib)

        def body(i, carry):
            c0 = 2 * i
            c1 = 2 * i + 1
            first = i == 0
            fire_phase(c0, idx_a, acc_a, sem_ia, sem_ga, sem_sa, first)
            fire_phase(c1, idx_b, acc_b, sem_ib, sem_gb, sem_sb, first)
            finish_phase(c0, idx_a, acc_a, sem_ia, sem_ga, sem_sa)
            finish_phase(c1, idx_b, acc_b, sem_ib, sem_gb, sem_sb)
            return carry

        lax.fori_loop(0, NCH // 2, body, 0)
        drain_store(acc_a, sem_sa)
        drain_store(acc_b, sem_sb)

    return kfn


def _mkidx(nbr, convs):
    rows = []
    for off_list, bg in convs:
        for tl, o in enumerate(off_list):
            grp = bg + tl // 4
            q = tl % 4
            rows.append(4 * nbr[:, o] + (4 * NP * grp + q))
    idx = jnp.stack(rows).astype(jnp.int32)
    k = idx.shape[0]
    return idx.reshape(k, NWORK, NCH, CH).transpose(1, 2, 0, 3)


def kernel(voxel_features, coors, neighbor_idx, W_c1, g0, b0, W_c12, g02, b02,
           W_c2, g1, b1, W_c3, g2, b2, Wr1, gr1, br1, Wr2, gr2, br2,
           Wr3, gr3, br3, W_logits):
    del coors
    r = 1.0 / math.sqrt(1.0 + EPS)
    x = voxel_features
    nbr = neighbor_idx

    Wa = jnp.concatenate([_flat_w(list(W_c1)), _flat_w(list(W_c2))], axis=1)
    Za = _tc1(x, Wa)
    Aa = _make_sc_gather(18, 2, 6)(Za.reshape(4 * 6 * NP, CMID), _mkidx(nbr, [(_K133, 0), (_K313, 3)]))

    P2 = jnp.stack([g0 * r, b0, g1 * r, b1]).reshape(4, 1, CMID)
    Zb = _tc2(Aa, _flat_w(list(W_c12)), _flat_w(list(W_c3)), P2)
    Ab = _make_sc_gather(18, 2, 6)(Zb.reshape(4 * 6 * NP, CMID), _mkidx(nbr, [(_K313, 0), (_K133, 3)]))

    P3 = jnp.stack([g02 * r, b02, g2 * r, b2]).reshape(4, 1, CMID)
    Wfr = jnp.concatenate(
        [_flat_w(list(Wr1)), _flat_w(list(Wr2)), _flat_w(list(Wr3))], axis=1)
    rA, Zc = _tc3(Ab, Wfr, P3)
    Rc = _make_sc_gather(9, 3, 3)(
        Zc.reshape(4 * 3 * NP, CMID), _mkidx(nbr, [(_K311, 0), (_K131, 1), (_K113, 2)]))

    P4 = jnp.stack([gr1 * r, br1, gr2 * r, br2, gr3 * r, br3]).reshape(6, 1, CMID)
    Wd = jnp.pad(W_logits, ((0, 0), (0, 0), (0, CMID - W_logits.shape[2])))
    Zd = _tc4(Rc, rA, _flat_w(list(Wd)), P4)
    L = _make_sc_gather(27, 1, 7)(Zd.reshape(4 * 7 * NP, CMID), _mkidx(nbr, [(_K333, 0)]))

    return L[:N, :W_logits.shape[2]].astype(F32)


# speedup vs baseline: 1.3882x; 1.3882x over previous
import functools
import itertools
import math

import jax
import jax.numpy as jnp
from jax import lax
from jax.experimental import pallas as pl
from jax.experimental.pallas import tpu as pltpu
from jax.experimental.pallas import tpu_sc as plsc

N = 65536
B = 1024
NP = N + B
NT = NP // B
CMID = 32
EPS = 1e-5

_OFFS = list(itertools.product([-1, 0, 1], repeat=3))


def _sel(pred):
    return [i for i, o in enumerate(_OFFS) if pred(o)]


_K133 = _sel(lambda o: o[0] == 0)
_K313 = _sel(lambda o: o[1] == 0)
_K311 = _sel(lambda o: o[1] == 0 and o[2] == 0)
_K131 = _sel(lambda o: o[0] == 0 and o[2] == 0)
_K113 = _sel(lambda o: o[0] == 0 and o[1] == 0)
_K333 = list(range(27))


def _lrelu(t):
    return jnp.maximum(t, 0.01 * t)


def _tc1(x, Wa):
    k = Wa.shape[0]

    def body(x_ref, w_ref, o_ref):
        i = pl.program_id(0)

        @pl.when(i < NT - 1)
        def _():
            xb = x_ref[...]
            for j in range(k):
                o_ref[j] = jnp.dot(xb, w_ref[j], preferred_element_type=jnp.float32).astype(jnp.bfloat16)

        @pl.when(i == NT - 1)
        def _():
            o_ref[...] = jnp.zeros_like(o_ref)

    return pl.pallas_call(
        body,
        grid=(NT,),
        in_specs=[
            pl.BlockSpec((B, 16), lambda i: (jnp.minimum(i, NT - 2), 0)),
            pl.BlockSpec((k, 16, CMID), lambda i: (0, 0, 0)),
        ],
        out_specs=pl.BlockSpec((k, B, CMID), lambda i: (0, i, 0)),
        out_shape=jax.ShapeDtypeStruct((k, NP, CMID), jnp.bfloat16),
    )(x, Wa)


def _tc2(Aa, W12, W3, P):
    def body(a_ref, w12_ref, w3_ref, p_ref, o_ref):
        i = pl.program_id(0)

        @pl.when(i < NT - 1)
        def _():
            u0 = _lrelu(a_ref[0].astype(jnp.float32)) * p_ref[0] + p_ref[1]
            u1 = _lrelu(a_ref[1].astype(jnp.float32)) * p_ref[2] + p_ref[3]
            for j in range(9):
                o_ref[j] = jnp.dot(u0, w12_ref[j], preferred_element_type=jnp.float32).astype(jnp.bfloat16)
                o_ref[9 + j] = jnp.dot(u1, w3_ref[j], preferred_element_type=jnp.float32).astype(jnp.bfloat16)

        @pl.when(i == NT - 1)
        def _():
            o_ref[...] = jnp.zeros_like(o_ref)

    return pl.pallas_call(
        body,
        grid=(NT,),
        in_specs=[
            pl.BlockSpec((2, B, CMID), lambda i: (0, i, 0)),
            pl.BlockSpec((9, CMID, CMID), lambda i: (0, 0, 0)),
            pl.BlockSpec((9, CMID, CMID), lambda i: (0, 0, 0)),
            pl.BlockSpec((4, 1, CMID), lambda i: (0, 0, 0)),
        ],
        out_specs=pl.BlockSpec((18, B, CMID), lambda i: (0, i, 0)),
        out_shape=jax.ShapeDtypeStruct((18, NP, CMID), jnp.bfloat16),
    )(Aa, W12, W3, P)


def _tc3(Ab, Wr, P):
    def body(a_ref, w_ref, p_ref, ra_ref, o_ref):
        i = pl.program_id(0)

        @pl.when(i < NT - 1)
        def _():
            rA = (_lrelu(a_ref[0].astype(jnp.float32)) * p_ref[0] + p_ref[1]) + (
                _lrelu(a_ref[1].astype(jnp.float32)) * p_ref[2] + p_ref[3])
            ra_ref[...] = rA
            for j in range(9):
                o_ref[j] = jnp.dot(rA, w_ref[j], preferred_element_type=jnp.float32).astype(jnp.bfloat16)

        @pl.when(i == NT - 1)
        def _():
            ra_ref[...] = jnp.zeros_like(ra_ref)
            o_ref[...] = jnp.zeros_like(o_ref)

    return pl.pallas_call(
        body,
        grid=(NT,),
        in_specs=[
            pl.BlockSpec((2, B, CMID), lambda i: (0, i, 0)),
            pl.BlockSpec((9, CMID, CMID), lambda i: (0, 0, 0)),
            pl.BlockSpec((4, 1, CMID), lambda i: (0, 0, 0)),
        ],
        out_specs=[
            pl.BlockSpec((B, CMID), lambda i: (i, 0)),
            pl.BlockSpec((9, B, CMID), lambda i: (0, i, 0)),
        ],
        out_shape=[
            jax.ShapeDtypeStruct((NP, CMID), jnp.float32),
            jax.ShapeDtypeStruct((9, NP, CMID), jnp.bfloat16),
        ],
    )(Ab, Wr, P)


def _tc4(Rc, rA, Wd, P):
    def body(r_ref, ra_ref, w_ref, p_ref, o_ref):
        i = pl.program_id(0)

        @pl.when(i < NT - 1)
        def _():
            s = (jax.nn.sigmoid(r_ref[0].astype(jnp.float32) * p_ref[0] + p_ref[1])
                 + jax.nn.sigmoid(r_ref[1].astype(jnp.float32) * p_ref[2] + p_ref[3])
                 + jax.nn.sigmoid(r_ref[2].astype(jnp.float32) * p_ref[4] + p_ref[5]))
            recon = s * ra_ref[...]
            for j in range(27):
                o_ref[j] = jnp.dot(recon, w_ref[j], preferred_element_type=jnp.float32).astype(jnp.bfloat16)

        @pl.when(i == NT - 1)
        def _():
            o_ref[...] = jnp.zeros_like(o_ref)

    return pl.pallas_call(
        body,
        grid=(NT,),
        in_specs=[
            pl.BlockSpec((3, B, CMID), lambda i: (0, i, 0)),
            pl.BlockSpec((B, CMID), lambda i: (i, 0)),
            pl.BlockSpec((27, CMID, CMID), lambda i: (0, 0, 0)),
            pl.BlockSpec((6, 1, CMID), lambda i: (0, 0, 0)),
        ],
        out_specs=pl.BlockSpec((27, B, CMID), lambda i: (0, i, 0)),
        out_shape=jax.ShapeDtypeStruct((27, NP, CMID), jnp.bfloat16),
    )(Rc, rA, Wd, P)


NWORK = 32
RW = N // NWORK
CH = 128
NCH = RW // CH


@functools.lru_cache(maxsize=None)
def _make_sc_gather(k, G):
    g = k // G
    mesh = plsc.VectorSubcoreMesh(core_axis_name="c", subcore_axis_name="s")

    @functools.partial(
        pl.kernel,
        out_type=jax.ShapeDtypeStruct((G, NP, CMID), jnp.bfloat16),
        mesh=mesh,
        scratch_types=[
            pltpu.VMEM((k, CH), jnp.int32),
            pltpu.VMEM((k, CH), jnp.int32),
            pltpu.VMEM((G, CH, CMID), jnp.bfloat16),
            pltpu.VMEM((G, CH, CMID), jnp.bfloat16),
            pltpu.SemaphoreType.DMA,
            pltpu.SemaphoreType.DMA,
            pltpu.SemaphoreType.DMA,
            pltpu.SemaphoreType.DMA,
            pltpu.SemaphoreType.DMA,
            pltpu.SemaphoreType.DMA,
        ],
        compiler_params=pltpu.CompilerParams(use_tc_tiling_on_sc=False),
    )
    def kfn(table, idx4, out, idx_a, idx_b, acc_a, acc_b,
            sem_ia, sem_ib, sem_ga, sem_gb, sem_sa, sem_sb):
        wid = lax.axis_index("s") * 2 + lax.axis_index("c")
        zero32 = jnp.zeros((CMID,), jnp.bfloat16)

        def zero_acc(acc):
            def zbody(r, carry):
                for grp in range(G):
                    acc[grp, r, :] = zero32
                return carry
            lax.fori_loop(0, CH, zbody, 0)

        def drain_store(acc, sem_s):
            for grp in range(G):
                pltpu.make_async_copy(
                    acc.at[grp], out.at[grp, pl.ds(0, CH)], sem_s).wait()

        def fire_phase(c, idx_v, acc, sem_i, sem_g, sem_s, first):
            @pl.when(jnp.logical_not(first))
            def _():
                drain_store(acc, sem_s)
            pltpu.make_async_copy(idx4.at[wid, 0], idx_v, sem_i).wait()
            zero_acc(acc)
            for grp in range(G):
                for j in range(g):
                    pltpu.async_copy(
                        table.at[idx_v.at[grp * g + j]], acc.at[grp], sem_g,
                        add=True)

        def finish_phase(c, idx_v, acc, sem_i, sem_g, sem_s):
            for grp in range(G):
                for j in range(g):
                    pltpu.make_async_copy(
                        table.at[idx_v.at[grp * g + j]], acc.at[grp],
                        sem_g).wait()
            base = wid * RW + c * CH
            for grp in range(G):
                pltpu.async_copy(acc.at[grp], out.at[grp, pl.ds(base, CH)], sem_s)
            @pl.when(c + 2 < NCH)
            def _():
                pltpu.async_copy(idx4.at[wid, c + 2], idx_v, sem_i)

        pltpu.async_copy(idx4.at[wid, 0], idx_a, sem_ia)
        pltpu.async_copy(idx4.at[wid, 1], idx_b, sem_ib)

        def body(i, carry):
            c0 = 2 * i
            c1 = 2 * i + 1
            first = i == 0
            fire_phase(c0, idx_a, acc_a, sem_ia, sem_ga, sem_sa, first)
            fire_phase(c1, idx_b, acc_b, sem_ib, sem_gb, sem_sb, first)
            finish_phase(c0, idx_a, acc_a, sem_ia, sem_ga, sem_sa)
            finish_phase(c1, idx_b, acc_b, sem_ib, sem_gb, sem_sb)
            return carry

        lax.fori_loop(0, NCH // 2, body, 0)
        drain_store(acc_a, sem_sa)
        drain_store(acc_b, sem_sb)

    return kfn


def _mkidx(nbr, offs):
    k = len(offs)
    cols = nbr[:, jnp.asarray(offs, dtype=jnp.int32)]
    base = (jnp.arange(k, dtype=jnp.int32) * NP)[:, None]
    idx = cols.T + base
    return idx.reshape(k, NWORK, NCH, CH).transpose(1, 2, 0, 3)


def kernel(voxel_features, coors, neighbor_idx, W_c1, g0, b0, W_c12, g02, b02,
           W_c2, g1, b1, W_c3, g2, b2, Wr1, gr1, br1, Wr2, gr2, br2,
           Wr3, gr3, br3, W_logits):
    del coors
    r = 1.0 / math.sqrt(1.0 + EPS)
    x = voxel_features
    nbr = neighbor_idx

    Wa = jnp.concatenate([W_c1, W_c2], axis=0)
    Za = _tc1(x, Wa)
    Aa = _make_sc_gather(18, 2)(Za.reshape(18 * NP, CMID), _mkidx(nbr, _K133 + _K313))

    P2 = jnp.stack([g0 * r, b0, g1 * r, b1]).reshape(4, 1, CMID)
    Zb = _tc2(Aa, W_c12, W_c3, P2)
    Ab = _make_sc_gather(18, 2)(Zb.reshape(18 * NP, CMID), _mkidx(nbr, _K313 + _K133))

    P3 = jnp.stack([g02 * r, b02, g2 * r, b2]).reshape(4, 1, CMID)
    Wr = jnp.concatenate([Wr1, Wr2, Wr3], axis=0)
    rA, Zc = _tc3(Ab, Wr, P3)
    Rc = _make_sc_gather(9, 3)(Zc.reshape(9 * NP, CMID), _mkidx(nbr, _K311 + _K131 + _K113))

    P4 = jnp.stack([gr1 * r, br1, gr2 * r, br2, gr3 * r, br3]).reshape(6, 1, CMID)
    Wd = jnp.pad(W_logits, ((0, 0), (0, 0), (0, CMID - W_logits.shape[2])))
    Zd = _tc4(Rc, rA, Wd, P4)
    L = _make_sc_gather(27, 1)(Zd.reshape(27 * NP, CMID), _mkidx(nbr, _K333))

    return L[0, :N, :W_logits.shape[2]].astype(jnp.float32)
